# baseline (speedup 1.0000x reference)
import functools
import math

import jax
import jax.numpy as jnp
from jax import lax
from jax.experimental import pallas as pl
from jax.experimental.pallas import tpu as pltpu
from jax.experimental.pallas import tpu_sc as plsc

_N = 10000
_E = 160000
_D = 256
_H = 8
_HD = _D // _H
_HALF = _D // 2

_BN = 1000
_GRID = _N // _BN

_NS = 16
_K = 64
_TPE = 10240
_EP = _NS * _TPE
_CHUNKS = _TPE // _K
_NPH = 4
_PCH = _CHUNKS // _NPH
_NBUF = 4
_ACC_R = 10240
_ZR = _ACC_R // _NS
_OR = 624


def _msg_math(x, wq_ref, wk_ref, wv_ref, wo_ref,
              bq_ref, bk_ref, bv_ref, bo_ref, bd_ref):
    q = jnp.dot(x, wq_ref[...], preferred_element_type=jnp.float32) + bq_ref[...]
    k = jnp.dot(x, wk_ref[...], preferred_element_type=jnp.float32) + bk_ref[...]
    v = jnp.dot(x, wv_ref[...], preferred_element_type=jnp.float32) + bv_ref[...]
    s = jnp.dot(q * k, bd_ref[...], preferred_element_type=jnp.float32)
    s = s * (1.0 / math.sqrt(_HD))
    mx = jnp.max(s, axis=-1, keepdims=True)
    e = jnp.exp(s - mx)
    den = jnp.sum(e, axis=-1, keepdims=True) * (1.0 / _HD)
    ao = (e / den) * v
    return jnp.dot(ao, wo_ref[...], preferred_element_type=jnp.float32) + bo_ref[...]


def _mlp_math(xm, msg_ref, lv, w1_ref, b1_ref, w2_ref, b2_ref,
              w3_ref, b3_ref, g_ref, be_ref):
    h = jnp.dot(xm, w1_ref[0:_D, :], preferred_element_type=jnp.float32)
    h = h + jnp.dot(msg_ref[0], w1_ref[_D:_D + _HALF, :],
                    preferred_element_type=jnp.float32)
    h = h + jnp.dot(msg_ref[1], w1_ref[_D + _HALF:, :],
                    preferred_element_type=jnp.float32)
    h = jnp.maximum(h + b1_ref[...], 0.0)
    h2 = jnp.dot(h, w2_ref[...], preferred_element_type=jnp.float32) + b2_ref[...]
    dp = jnp.dot(h2, w3_ref[...], preferred_element_type=jnp.float32) + b3_ref[...]
    nm = dp[:, :_D]
    nl = dp[:, _D:]
    pre = nm + xm
    mu = jnp.mean(pre, axis=-1, keepdims=True)
    var = jnp.mean((pre - mu) ** 2, axis=-1, keepdims=True)
    nmo = (pre - mu) / jnp.sqrt(var + 1e-5) * g_ref[...] + be_ref[...]
    kl = 0.5 * (nl - lv + (jnp.exp(lv) + (xm - nmo) ** 2) / jnp.exp(nl) - 1.0)
    return nmo, nl, kl


def _msg_body(x_ref, wq_ref, wk_ref, wv_ref, wo_ref,
              bq_ref, bk_ref, bv_ref, bo_ref, bd_ref, out_ref):
    m = _msg_math(x_ref[...], wq_ref, wk_ref, wv_ref, wo_ref,
                  bq_ref, bk_ref, bv_ref, bo_ref, bd_ref)
    out_ref[0] = m[:, :_HALF]
    out_ref[1] = m[:, _HALF:]


def _mlp_body(mean_ref, msg_ref, lv_ref, w1_ref, b1_ref,
              w2_ref, b2_ref, w3_ref, b3_ref, g_ref, be_ref,
              nm_ref, nl_ref, kl_ref):
    nmo, nl, kl = _mlp_math(mean_ref[...], msg_ref, lv_ref[...],
                            w1_ref, b1_ref, w2_ref, b2_ref,
                            w3_ref, b3_ref, g_ref, be_ref)
    nm_ref[...] = nmo
    nl_ref[...] = nl

    @pl.when(pl.program_id(0) == 0)
    def _init():
        kl_ref[0, 0] = 0.0

    kl_ref[0, 0] += jnp.sum(kl) * (1.0 / _N)


def _fused_body(mean_ref, msg_ref, lv_ref, w1_ref, b1_ref,
                w2_ref, b2_ref, w3_ref, b3_ref, g_ref, be_ref,
                wq_ref, wk_ref, wv_ref, wo_ref,
                bq_ref, bk_ref, bv_ref, bo_ref, bd_ref,
                nm_ref, nl_ref, kl_ref, mtab_ref):
    nmo, nl, kl = _mlp_math(mean_ref[...], msg_ref, lv_ref[...],
                            w1_ref, b1_ref, w2_ref, b2_ref,
                            w3_ref, b3_ref, g_ref, be_ref)
    nm_ref[...] = nmo
    nl_ref[...] = nl

    @pl.when(pl.program_id(0) == 0)
    def _init():
        kl_ref[0, 0] = 0.0

    kl_ref[0, 0] += jnp.sum(kl) * (1.0 / _N)

    m = _msg_math(nmo, wq_ref, wk_ref, wv_ref, wo_ref,
                  bq_ref, bk_ref, bv_ref, bo_ref, bd_ref)
    mtab_ref[0] = m[:, :_HALF]
    mtab_ref[1] = m[:, _HALF:]


def _scatter_body(mtab, idx, zeros, out, ibuf, rows, acc,
                  g0, g1, g2, g3, t0, t1, t2, t3):
    c = lax.axis_index("c")
    s = lax.axis_index("s")
    gsem = (g0, g1, g2, g3)
    ssem = (t0, t1, t2, t3)
    pltpu.sync_copy(zeros, acc.at[pl.ds(s * _ZR, _ZR)])
    plsc.subcore_barrier()

    def gather(j, b):
        pltpu.async_copy(mtab.at[ibuf.at[j, 0]], rows.at[b], gsem[b])

    def gwait(b):
        pltpu.make_async_copy(mtab.at[ibuf.at[0, 0]], rows.at[b],
                              gsem[b]).wait()

    def scat(j, b):
        pltpu.async_copy(rows.at[b], acc.at[ibuf.at[j, 1]], ssem[b], add=True)

    def swait(b):
        pltpu.make_async_copy(rows.at[b], acc.at[ibuf.at[0, 1]],
                              ssem[b]).wait()

    for ph in range(_NPH):
        pltpu.sync_copy(idx.at[c, s, ph], ibuf)
        gather(0, 0)
        gather(1, 1)
        gwait(0); scat(0, 0); gather(2, 2)
        gwait(1); scat(1, 1); gather(3, 3)
        gwait(2); scat(2, 2); swait(0); gather(4, 0)
        gwait(3); scat(3, 3); swait(1); gather(5, 1)

        def body(i, carry):
            for b in range(_NBUF):
                j = (i + 1) * _NBUF + b
                gwait(b)
                scat(j, b)
                nb = (b + 2) % _NBUF
                swait(nb)
                gather(j + 2, nb)
            return carry

        lax.fori_loop(0, _PCH // _NBUF - 2, body, 0)
        jb = _PCH - _NBUF
        gwait(0); scat(jb, 0); swait(2); gather(jb + 2, 2)
        gwait(1); scat(jb + 1, 1); swait(3); gather(jb + 3, 3)
        gwait(2); scat(jb + 2, 2)
        gwait(3); scat(jb + 3, 3)
        for b in range(_NBUF):
            swait(b)
    plsc.subcore_barrier()
    pltpu.sync_copy(acc.at[pl.ds(s * _OR, _OR)], out.at[c, pl.ds(s * _OR, _OR)])

    @pl.when(s == _NS - 1)
    def _tail():
        pltpu.sync_copy(acc.at[pl.ds(_NS * _OR, _N - _NS * _OR)],
                        out.at[c, pl.ds(_NS * _OR, _N - _NS * _OR)])


_row_spec = pl.BlockSpec((_BN, _D), lambda i: (i, 0))
_full = lambda shape: pl.BlockSpec(shape, lambda i: tuple(0 for _ in shape))

_msg_call = pl.pallas_call(
    _msg_body,
    grid=(_GRID,),
    in_specs=[
        _row_spec,
        _full((_D, _D)), _full((_D, _D)), _full((_D, _D)), _full((_D, _D)),
        _full((1, _D)), _full((1, _D)), _full((1, _D)), _full((1, _D)),
        _full((_D, _D)),
    ],
    out_specs=pl.BlockSpec((2, _BN, _HALF), lambda i: (0, i, 0)),
    out_shape=jax.ShapeDtypeStruct((2, _N, _HALF), jnp.float32),
)

_mlp_call = pl.pallas_call(
    _mlp_body,
    grid=(_GRID,),
    in_specs=[
        _row_spec,
        pl.BlockSpec((2, _BN, _HALF), lambda i: (0, i, 0)),
        _row_spec,
        _full((2 * _D, 4 * _D)), _full((1, 4 * _D)),
        _full((4 * _D, 2 * _D)), _full((1, 2 * _D)),
        _full((2 * _D, 2 * _D)), _full((1, 2 * _D)),
        _full((1, _D)), _full((1, _D)),
    ],
    out_specs=[
        _row_spec,
        _row_spec,
        pl.BlockSpec((1, 1), lambda i: (0, 0), memory_space=pltpu.SMEM),
    ],
    out_shape=[
        jax.ShapeDtypeStruct((_N, _D), jnp.float32),
        jax.ShapeDtypeStruct((_N, _D), jnp.float32),
        jax.ShapeDtypeStruct((1, 1), jnp.float32),
    ],
    compiler_params=pltpu.CompilerParams(
        dimension_semantics=("arbitrary",)),
)

_fused_call = pl.pallas_call(
    _fused_body,
    grid=(_GRID,),
    in_specs=[
        _row_spec,
        pl.BlockSpec((2, _BN, _HALF), lambda i: (0, i, 0)),
        _row_spec,
        _full((2 * _D, 4 * _D)), _full((1, 4 * _D)),
        _full((4 * _D, 2 * _D)), _full((1, 2 * _D)),
        _full((2 * _D, 2 * _D)), _full((1, 2 * _D)),
        _full((1, _D)), _full((1, _D)),
        _full((_D, _D)), _full((_D, _D)), _full((_D, _D)), _full((_D, _D)),
        _full((1, _D)), _full((1, _D)), _full((1, _D)), _full((1, _D)),
        _full((_D, _D)),
    ],
    out_specs=[
        _row_spec,
        _row_spec,
        pl.BlockSpec((1, 1), lambda i: (0, 0), memory_space=pltpu.SMEM),
        pl.BlockSpec((2, _BN, _HALF), lambda i: (0, i, 0)),
    ],
    out_shape=[
        jax.ShapeDtypeStruct((_N, _D), jnp.float32),
        jax.ShapeDtypeStruct((_N, _D), jnp.float32),
        jax.ShapeDtypeStruct((1, 1), jnp.float32),
        jax.ShapeDtypeStruct((2, _N, _HALF), jnp.float32),
    ],
    compiler_params=pltpu.CompilerParams(
        dimension_semantics=("arbitrary",)),
)

@functools.cache
def _get_scatter_call():
    return pl.kernel(
        _scatter_body,
        out_type=jax.ShapeDtypeStruct((2, _N, _HALF), jnp.float32),
        mesh=plsc.VectorSubcoreMesh(core_axis_name="c", subcore_axis_name="s"),
        scratch_types=[
            pltpu.VMEM((_PCH, 2, _K), jnp.int32),
            pltpu.VMEM((_NBUF, _K, _HALF), jnp.float32),
            pltpu.VMEM_SHARED((_ACC_R, _HALF), jnp.float32),
            pltpu.SemaphoreType.DMA,
            pltpu.SemaphoreType.DMA,
            pltpu.SemaphoreType.DMA,
            pltpu.SemaphoreType.DMA,
            pltpu.SemaphoreType.DMA,
            pltpu.SemaphoreType.DMA,
            pltpu.SemaphoreType.DMA,
            pltpu.SemaphoreType.DMA,
        ],
    )


def kernel(x, edge_index, params):
    src = edge_index[0].astype(jnp.int32)
    dst = edge_index[1].astype(jnp.int32)
    pad = _EP - _E
    src_pad = jnp.concatenate([src, jnp.zeros((pad,), jnp.int32)])
    dst_pad = jnp.concatenate([dst, jnp.full((pad,), _N, jnp.int32)])
    srcs = jnp.stack(
        [src_pad, src_pad + _N]).reshape(2, _NS, _NPH, _PCH, _K)
    dsts = jnp.broadcast_to(
        dst_pad, (2, _EP)).reshape(2, _NS, _NPH, _PCH, _K)
    idx = jnp.stack([srcs, dsts], axis=4)
    zeros = jnp.zeros((_ZR, _HALF), jnp.float32)
    row = lax.broadcasted_iota(jnp.int32, (_D, _D), 0) // _HD
    col = lax.broadcasted_iota(jnp.int32, (_D, _D), 1) // _HD
    bdiag = (row == col).astype(jnp.float32)

    mean = x
    logvar = jnp.zeros((_N, _D), jnp.float32)
    total_kl = jnp.float32(0.0)
    nl = len(params)
    p0 = params[0]
    mtab = _msg_call(
        mean, p0['Wq'], p0['Wk'], p0['Wv'], p0['Wo'],
        p0['bq'].reshape(1, _D), p0['bk'].reshape(1, _D),
        p0['bv'].reshape(1, _D), p0['bo'].reshape(1, _D), bdiag)
    for l, p in enumerate(params):
        msgs = _get_scatter_call()(mtab.reshape(2 * _N, _HALF), idx, zeros)
        mlp_args = (
            mean, msgs, logvar,
            p['W1'], p['b1'].reshape(1, 4 * _D),
            p['W2'], p['b2'].reshape(1, 2 * _D),
            p['W3'], p['b3'].reshape(1, 2 * _D),
            p['gamma'].reshape(1, _D), p['beta'].reshape(1, _D))
        if l + 1 < nl:
            pn = params[l + 1]
            mean, logvar, klp, mtab = _fused_call(
                *mlp_args,
                pn['Wq'], pn['Wk'], pn['Wv'], pn['Wo'],
                pn['bq'].reshape(1, _D), pn['bk'].reshape(1, _D),
                pn['bv'].reshape(1, _D), pn['bo'].reshape(1, _D), bdiag)
        else:
            mean, logvar, klp = _mlp_call(*mlp_args)
        total_kl = total_kl + klp[0, 0]
    return mean, mean, logvar, total_kl


# speedup vs baseline: 1.0537x; 1.0537x over previous
import functools
import math

import jax
import jax.numpy as jnp
from jax import lax
from jax.experimental import pallas as pl
from jax.experimental.pallas import tpu as pltpu
from jax.experimental.pallas import tpu_sc as plsc

_N = 10000
_E = 160000
_D = 256
_H = 8
_HD = _D // _H
_HALF = _D // 2

_BN = 2000
_GRID = _N // _BN

_NS = 16
_K = 128
_TPE = 10240
_EP = _NS * _TPE
_CHUNKS = _TPE // _K
_NPH = 8
_PCH = _CHUNKS // _NPH
_NBUF = 2
_ACC_R = 10240
_ZR = _ACC_R // _NS
_OR = 624


def _msg_math(x, wq_ref, wk_ref, wv_ref, wo_ref,
              bq_ref, bk_ref, bv_ref, bo_ref, bd_ref):
    q = jnp.dot(x, wq_ref[...], preferred_element_type=jnp.float32) + bq_ref[...]
    k = jnp.dot(x, wk_ref[...], preferred_element_type=jnp.float32) + bk_ref[...]
    v = jnp.dot(x, wv_ref[...], preferred_element_type=jnp.float32) + bv_ref[...]
    s = jnp.dot(q * k, bd_ref[...], preferred_element_type=jnp.float32)
    s = s * (1.0 / math.sqrt(_HD))
    mx = jnp.max(s, axis=-1, keepdims=True)
    e = jnp.exp(s - mx)
    den = jnp.sum(e, axis=-1, keepdims=True) * (1.0 / _HD)
    ao = (e / den) * v
    return jnp.dot(ao, wo_ref[...], preferred_element_type=jnp.float32) + bo_ref[...]


def _mlp_math(xm, msg_ref, lv, w1_ref, b1_ref, w2_ref, b2_ref,
              w3_ref, b3_ref, g_ref, be_ref):
    h = jnp.dot(xm, w1_ref[0:_D, :], preferred_element_type=jnp.float32)
    h = h + jnp.dot(msg_ref[0], w1_ref[_D:_D + _HALF, :],
                    preferred_element_type=jnp.float32)
    h = h + jnp.dot(msg_ref[1], w1_ref[_D + _HALF:, :],
                    preferred_element_type=jnp.float32)
    h = jnp.maximum(h + b1_ref[...], 0.0)
    h2 = jnp.dot(h, w2_ref[...], preferred_element_type=jnp.float32) + b2_ref[...]
    dp = jnp.dot(h2, w3_ref[...], preferred_element_type=jnp.float32) + b3_ref[...]
    nm = dp[:, :_D]
    nl = dp[:, _D:]
    pre = nm + xm
    mu = jnp.mean(pre, axis=-1, keepdims=True)
    var = jnp.mean((pre - mu) ** 2, axis=-1, keepdims=True)
    nmo = (pre - mu) / jnp.sqrt(var + 1e-5) * g_ref[...] + be_ref[...]
    kl = 0.5 * (nl - lv + (jnp.exp(lv) + (xm - nmo) ** 2) / jnp.exp(nl) - 1.0)
    return nmo, nl, kl


def _msg_body(x_ref, wq_ref, wk_ref, wv_ref, wo_ref,
              bq_ref, bk_ref, bv_ref, bo_ref, bd_ref, out_ref):
    m = _msg_math(x_ref[...], wq_ref, wk_ref, wv_ref, wo_ref,
                  bq_ref, bk_ref, bv_ref, bo_ref, bd_ref)
    out_ref[0] = m[:, :_HALF]
    out_ref[1] = m[:, _HALF:]


def _mlp_body(mean_ref, msg_ref, lv_ref, w1_ref, b1_ref,
              w2_ref, b2_ref, w3_ref, b3_ref, g_ref, be_ref,
              nm_ref, nl_ref, kl_ref):
    nmo, nl, kl = _mlp_math(mean_ref[...], msg_ref, lv_ref[...],
                            w1_ref, b1_ref, w2_ref, b2_ref,
                            w3_ref, b3_ref, g_ref, be_ref)
    nm_ref[...] = nmo
    nl_ref[...] = nl

    @pl.when(pl.program_id(0) == 0)
    def _init():
        kl_ref[0, 0] = 0.0

    kl_ref[0, 0] += jnp.sum(kl) * (1.0 / _N)


def _fused_body(mean_ref, msg_ref, lv_ref, w1_ref, b1_ref,
                w2_ref, b2_ref, w3_ref, b3_ref, g_ref, be_ref,
                wq_ref, wk_ref, wv_ref, wo_ref,
                bq_ref, bk_ref, bv_ref, bo_ref, bd_ref,
                nm_ref, nl_ref, kl_ref, mtab_ref):
    nmo, nl, kl = _mlp_math(mean_ref[...], msg_ref, lv_ref[...],
                            w1_ref, b1_ref, w2_ref, b2_ref,
                            w3_ref, b3_ref, g_ref, be_ref)
    nm_ref[...] = nmo
    nl_ref[...] = nl

    @pl.when(pl.program_id(0) == 0)
    def _init():
        kl_ref[0, 0] = 0.0

    kl_ref[0, 0] += jnp.sum(kl) * (1.0 / _N)

    m = _msg_math(nmo, wq_ref, wk_ref, wv_ref, wo_ref,
                  bq_ref, bk_ref, bv_ref, bo_ref, bd_ref)
    mtab_ref[0] = m[:, :_HALF]
    mtab_ref[1] = m[:, _HALF:]


def _scatter_body(mtab, idx, zeros, out, ibuf, rows, acc, g0, g1, g2, g3):
    c = lax.axis_index("c")
    s = lax.axis_index("s")
    gsem = (g0, g1, g2, g3)
    pltpu.sync_copy(zeros, acc.at[pl.ds(s * _ZR, _ZR)])
    plsc.subcore_barrier()

    def gather(j, b):
        pltpu.async_copy(mtab.at[ibuf.at[j, 0]], rows.at[b], gsem[b])

    def gwait(b):
        pltpu.make_async_copy(mtab.at[ibuf.at[0, 0]], rows.at[b],
                              gsem[b]).wait()

    for ph in range(_NPH):
        pltpu.sync_copy(idx.at[c, s, ph], ibuf)
        for b in range(_NBUF):
            gather(b, b)

        def body(i, carry):
            for b in range(_NBUF):
                j = i * _NBUF + b
                gwait(b)
                pltpu.sync_copy(rows.at[b], acc.at[ibuf.at[j, 1]], add=True)
                gather(j + _NBUF, b)
            return carry

        lax.fori_loop(0, _PCH // _NBUF - 1, body, 0)
        for b in range(_NBUF):
            j = _PCH - _NBUF + b
            gwait(b)
            pltpu.sync_copy(rows.at[b], acc.at[ibuf.at[j, 1]], add=True)
    plsc.subcore_barrier()
    pltpu.sync_copy(acc.at[pl.ds(s * _OR, _OR)], out.at[c, pl.ds(s * _OR, _OR)])

    @pl.when(s == _NS - 1)
    def _tail():
        pltpu.sync_copy(acc.at[pl.ds(_NS * _OR, _N - _NS * _OR)],
                        out.at[c, pl.ds(_NS * _OR, _N - _NS * _OR)])


_row_spec = pl.BlockSpec((_BN, _D), lambda i: (i, 0))
_full = lambda shape: pl.BlockSpec(shape, lambda i: tuple(0 for _ in shape))

_msg_call = pl.pallas_call(
    _msg_body,
    grid=(_GRID,),
    in_specs=[
        _row_spec,
        _full((_D, _D)), _full((_D, _D)), _full((_D, _D)), _full((_D, _D)),
        _full((1, _D)), _full((1, _D)), _full((1, _D)), _full((1, _D)),
        _full((_D, _D)),
    ],
    out_specs=pl.BlockSpec((2, _BN, _HALF), lambda i: (0, i, 0)),
    out_shape=jax.ShapeDtypeStruct((2, _N, _HALF), jnp.float32),
)

_mlp_call = pl.pallas_call(
    _mlp_body,
    grid=(_GRID,),
    in_specs=[
        _row_spec,
        pl.BlockSpec((2, _BN, _HALF), lambda i: (0, i, 0)),
        _row_spec,
        _full((2 * _D, 4 * _D)), _full((1, 4 * _D)),
        _full((4 * _D, 2 * _D)), _full((1, 2 * _D)),
        _full((2 * _D, 2 * _D)), _full((1, 2 * _D)),
        _full((1, _D)), _full((1, _D)),
    ],
    out_specs=[
        _row_spec,
        _row_spec,
        pl.BlockSpec((1, 1), lambda i: (0, 0), memory_space=pltpu.SMEM),
    ],
    out_shape=[
        jax.ShapeDtypeStruct((_N, _D), jnp.float32),
        jax.ShapeDtypeStruct((_N, _D), jnp.float32),
        jax.ShapeDtypeStruct((1, 1), jnp.float32),
    ],
    compiler_params=pltpu.CompilerParams(
        dimension_semantics=("arbitrary",)),
)

_fused_call = pl.pallas_call(
    _fused_body,
    grid=(_GRID,),
    in_specs=[
        _row_spec,
        pl.BlockSpec((2, _BN, _HALF), lambda i: (0, i, 0)),
        _row_spec,
        _full((2 * _D, 4 * _D)), _full((1, 4 * _D)),
        _full((4 * _D, 2 * _D)), _full((1, 2 * _D)),
        _full((2 * _D, 2 * _D)), _full((1, 2 * _D)),
        _full((1, _D)), _full((1, _D)),
        _full((_D, _D)), _full((_D, _D)), _full((_D, _D)), _full((_D, _D)),
        _full((1, _D)), _full((1, _D)), _full((1, _D)), _full((1, _D)),
        _full((_D, _D)),
    ],
    out_specs=[
        _row_spec,
        _row_spec,
        pl.BlockSpec((1, 1), lambda i: (0, 0), memory_space=pltpu.SMEM),
        pl.BlockSpec((2, _BN, _HALF), lambda i: (0, i, 0)),
    ],
    out_shape=[
        jax.ShapeDtypeStruct((_N, _D), jnp.float32),
        jax.ShapeDtypeStruct((_N, _D), jnp.float32),
        jax.ShapeDtypeStruct((1, 1), jnp.float32),
        jax.ShapeDtypeStruct((2, _N, _HALF), jnp.float32),
    ],
    compiler_params=pltpu.CompilerParams(
        dimension_semantics=("arbitrary",)),
)

@functools.cache
def _get_scatter_call():
    return pl.kernel(
        _scatter_body,
        out_type=jax.ShapeDtypeStruct((2, _N, _HALF), jnp.float32),
        mesh=plsc.VectorSubcoreMesh(core_axis_name="c", subcore_axis_name="s"),
        scratch_types=[
            pltpu.VMEM((_PCH, 2, _K), jnp.int32),
            pltpu.VMEM((_NBUF, _K, _HALF), jnp.float32),
            pltpu.VMEM_SHARED((_ACC_R, _HALF), jnp.float32),
            pltpu.SemaphoreType.DMA,
            pltpu.SemaphoreType.DMA,
            pltpu.SemaphoreType.DMA,
            pltpu.SemaphoreType.DMA,
        ],
    )


def kernel(x, edge_index, params):
    src = edge_index[0].astype(jnp.int32)
    dst = edge_index[1].astype(jnp.int32)
    pad = _EP - _E
    src_pad = jnp.concatenate([src, jnp.zeros((pad,), jnp.int32)])
    dst_pad = jnp.concatenate([dst, jnp.full((pad,), _N, jnp.int32)])
    srcs = jnp.stack(
        [src_pad, src_pad + _N]).reshape(2, _NS, _NPH, _PCH, _K)
    dsts = jnp.broadcast_to(
        dst_pad, (2, _EP)).reshape(2, _NS, _NPH, _PCH, _K)
    idx = jnp.stack([srcs, dsts], axis=4)
    zeros = jnp.zeros((_ZR, _HALF), jnp.float32)
    row = lax.broadcasted_iota(jnp.int32, (_D, _D), 0) // _HD
    col = lax.broadcasted_iota(jnp.int32, (_D, _D), 1) // _HD
    bdiag = (row == col).astype(jnp.float32)

    mean = x
    logvar = jnp.zeros((_N, _D), jnp.float32)
    total_kl = jnp.float32(0.0)
    nl = len(params)
    p0 = params[0]
    mtab = _msg_call(
        mean, p0['Wq'], p0['Wk'], p0['Wv'], p0['Wo'],
        p0['bq'].reshape(1, _D), p0['bk'].reshape(1, _D),
        p0['bv'].reshape(1, _D), p0['bo'].reshape(1, _D), bdiag)
    for l, p in enumerate(params):
        msgs = _get_scatter_call()(mtab.reshape(2 * _N, _HALF), idx, zeros)
        mlp_args = (
            mean, msgs, logvar,
            p['W1'], p['b1'].reshape(1, 4 * _D),
            p['W2'], p['b2'].reshape(1, 2 * _D),
            p['W3'], p['b3'].reshape(1, 2 * _D),
            p['gamma'].reshape(1, _D), p['beta'].reshape(1, _D))
        if l + 1 < nl:
            pn = params[l + 1]
            mean, logvar, klp, mtab = _fused_call(
                *mlp_args,
                pn['Wq'], pn['Wk'], pn['Wv'], pn['Wo'],
                pn['bq'].reshape(1, _D), pn['bk'].reshape(1, _D),
                pn['bv'].reshape(1, _D), pn['bo'].reshape(1, _D), bdiag)
        else:
            mean, logvar, klp = _mlp_call(*mlp_args)
        total_kl = total_kl + klp[0, 0]
    return mean, mean, logvar, total_kl
